# Initial kernel scaffold; baseline (speedup 1.0000x reference)
import jax
import jax.numpy as jnp
from jax.experimental import pallas as pl


def kernel(x, weight):
    raise NotImplementedError("write your pallas kernel here")


# speedup vs baseline: 1.2454x; 1.2454x over previous
import functools

import jax
import jax.numpy as jnp
from jax import lax
from jax.experimental import pallas as pl
from jax.experimental.pallas import tpu as pltpu
from jax.experimental.pallas import tpu_sc as plsc

EMBED_DIM = 300
PAD_DIM = 384
BATCH = 4096
SEQ = 200
B_TOTAL = BATCH * SEQ
NUM_WORKERS = 32
B_PER_W = B_TOTAL // NUM_WORKERS
CHUNK = 128
N_CHUNKS = B_PER_W // CHUNK

_mesh = plsc.VectorSubcoreMesh(core_axis_name="c", subcore_axis_name="s")


@functools.partial(
    pl.kernel,
    mesh=_mesh,
    out_type=jax.ShapeDtypeStruct((B_TOTAL, PAD_DIM), jnp.float32),
    scratch_types=[
        pltpu.VMEM((CHUNK,), jnp.int32),
        pltpu.VMEM((CHUNK, PAD_DIM), jnp.float32),
        pltpu.SemaphoreType.DMA,
    ],
)
def _embed_gather(idx_hbm, tab_hbm, out_hbm, idx_v, rows_v, sem):
    wid = lax.axis_index("s") * 2 + lax.axis_index("c")
    wbase = wid * B_PER_W

    def body(g, carry):
        base = wbase + g * CHUNK
        pltpu.sync_copy(idx_hbm.at[pl.ds(base, CHUNK)], idx_v)
        pltpu.async_copy(tab_hbm.at[idx_v], rows_v, sem).wait()
        pltpu.sync_copy(rows_v, out_hbm.at[pl.ds(base, CHUNK)])
        return carry

    lax.fori_loop(0, N_CHUNKS, body, 0)


def kernel(x, weight):
    idx = x.reshape(-1).astype(jnp.int32)
    w_pad = jnp.pad(weight, ((0, 0), (0, PAD_DIM - EMBED_DIM)))
    out = _embed_gather(idx, w_pad)
    return out[:, :EMBED_DIM].reshape(BATCH, SEQ, EMBED_DIM)


# speedup vs baseline: 1.5143x; 1.2159x over previous
import functools

import jax
import jax.numpy as jnp
from jax import lax
from jax.experimental import pallas as pl
from jax.experimental.pallas import tpu as pltpu
from jax.experimental.pallas import tpu_sc as plsc

EMBED_DIM = 300
PAD_DIM = 384
BATCH = 4096
SEQ = 200
B_TOTAL = BATCH * SEQ
NUM_WORKERS = 32
B_PER_W = B_TOTAL // NUM_WORKERS
CHUNK = 128
N_CHUNKS = B_PER_W // CHUNK

_mesh = plsc.VectorSubcoreMesh(core_axis_name="c", subcore_axis_name="s")


@functools.partial(
    pl.kernel,
    mesh=_mesh,
    out_type=jax.ShapeDtypeStruct((B_TOTAL, EMBED_DIM), jnp.float32),
    scratch_types=[
        pltpu.VMEM((CHUNK,), jnp.int32),
        pltpu.VMEM((CHUNK, PAD_DIM), jnp.float32),
        pltpu.SemaphoreType.DMA,
    ],
)
def _embed_gather(idx_hbm, tab_hbm, out_hbm, idx_v, rows_v, sem):
    wid = lax.axis_index("s") * 2 + lax.axis_index("c")
    wbase = wid * B_PER_W

    def body(g, carry):
        base = wbase + g * CHUNK
        pltpu.sync_copy(idx_hbm.at[pl.ds(base, CHUNK)], idx_v)
        pltpu.async_copy(
            tab_hbm.at[:, pl.ds(0, PAD_DIM)].at[idx_v], rows_v, sem).wait()
        pltpu.sync_copy(rows_v,
                        out_hbm.at[pl.ds(base, CHUNK), pl.ds(0, PAD_DIM)])
        return carry

    lax.fori_loop(0, N_CHUNKS, body, 0)


def kernel(x, weight):
    idx = x.reshape(-1).astype(jnp.int32)
    out = _embed_gather(idx, weight)
    return out.reshape(BATCH, SEQ, EMBED_DIM)


# speedup vs baseline: 1.6945x; 1.1190x over previous
import functools

import jax
import jax.numpy as jnp
from jax import lax
from jax.experimental import pallas as pl
from jax.experimental.pallas import tpu as pltpu
from jax.experimental.pallas import tpu_sc as plsc

EMBED_DIM = 300
PAD_DIM = 384
BATCH = 4096
SEQ = 200
B_TOTAL = BATCH * SEQ
NUM_WORKERS = 32
B_PER_W = B_TOTAL // NUM_WORKERS
CHUNK = 128
N_CHUNKS = B_PER_W // CHUNK
N_PAIRS = N_CHUNKS // 2

_mesh = plsc.VectorSubcoreMesh(core_axis_name="c", subcore_axis_name="s")


@functools.partial(
    pl.kernel,
    mesh=_mesh,
    out_type=jax.ShapeDtypeStruct((B_TOTAL, EMBED_DIM), jnp.float32),
    scratch_types=[
        pltpu.VMEM((2, CHUNK), jnp.int32),
        pltpu.VMEM((CHUNK, PAD_DIM), jnp.float32),
        pltpu.VMEM((CHUNK, PAD_DIM), jnp.float32),
        pltpu.SemaphoreType.DMA,
        pltpu.SemaphoreType.DMA,
        pltpu.SemaphoreType.DMA,
        pltpu.SemaphoreType.DMA,
    ],
)
def _embed_gather(idx_hbm, tab_hbm, out_hbm, idx_v, rows0, rows1, g0, g1,
                  o0, o1):
    wid = lax.axis_index("s") * 2 + lax.axis_index("c")
    wbase = wid * B_PER_W
    rows = (rows0, rows1)
    gsem = (g0, g1)
    osem = (o0, o1)
    tabp = tab_hbm.at[:, pl.ds(0, PAD_DIM)]

    def out_slice(g):
        return out_hbm.at[pl.ds(wbase + g * CHUNK, CHUNK), pl.ds(0, PAD_DIM)]

    def start_gather(g, k):
        pltpu.sync_copy(idx_hbm.at[pl.ds(wbase + g * CHUNK, CHUNK)],
                        idx_v.at[k])
        pltpu.async_copy(tabp.at[idx_v.at[k]], rows[k], gsem[k])

    def start_store(g, k):
        pltpu.make_async_copy(tabp.at[idx_v.at[k]], rows[k], gsem[k]).wait()
        pltpu.async_copy(rows[k], out_slice(g), osem[k])

    def wait_store(g, k):
        pltpu.make_async_copy(rows[k], out_slice(g), osem[k]).wait()

    start_gather(0, 0)

    def body(p, carry):
        a = 2 * p
        b = a + 1

        @pl.when(p > 0)
        def _():
            wait_store(b - 2, 1)

        start_gather(b, 1)
        start_store(a, 0)
        wait_store(a, 0)

        @pl.when(p < N_PAIRS - 1)
        def _():
            start_gather(a + 2, 0)

        start_store(b, 1)
        return carry

    lax.fori_loop(0, N_PAIRS, body, 0)
    wait_store(N_CHUNKS - 1, 1)


def kernel(x, weight):
    idx = x.reshape(-1).astype(jnp.int32)
    out = _embed_gather(idx, weight)
    return out.reshape(BATCH, SEQ, EMBED_DIM)
